# baseline (speedup 1.0000x reference)
import jax
import jax.numpy as jnp
from jax.experimental import pallas as pl
from jax.experimental.pallas import tpu as pltpu

_N_CAND = 8
_ROWS = 2 * 2048
_D = 1024
_NCHUNKS = 8
_CHUNK = _ROWS // _NCHUNKS


def _dma_body(s_ref, in_ref, out_ref, sems):
    s = s_ref[0]
    for i in range(_NCHUNKS):
        pltpu.make_async_copy(
            in_ref.at[s, pl.ds(i * _CHUNK, _CHUNK), :],
            out_ref.at[pl.ds(i * _CHUNK, _CHUNK), :],
            sems.at[i],
        ).start()
    for i in range(_NCHUNKS):
        pltpu.make_async_copy(
            in_ref.at[s, pl.ds(i * _CHUNK, _CHUNK), :],
            out_ref.at[pl.ds(i * _CHUNK, _CHUNK), :],
            sems.at[i],
        ).wait()


def kernel(inputs, binary_gates, alpha, sampled):
    del binary_gates, alpha
    s = jnp.asarray(sampled, dtype=jnp.int32).reshape((1,))
    flat = inputs.reshape(_N_CAND, _ROWS, _D)
    out = pl.pallas_call(
        _dma_body,
        grid_spec=pltpu.PrefetchScalarGridSpec(
            num_scalar_prefetch=1,
            in_specs=[pl.BlockSpec(memory_space=pl.ANY)],
            out_specs=pl.BlockSpec(memory_space=pl.ANY),
            scratch_shapes=[pltpu.SemaphoreType.DMA((_NCHUNKS,))],
        ),
        out_shape=jax.ShapeDtypeStruct((_ROWS, _D), jnp.float32),
    )(s, flat)
    return out.reshape(2, 2048, _D)


# speedup vs baseline: 43.9564x; 43.9564x over previous
import jax
import jax.numpy as jnp
from jax.experimental import pallas as pl
from jax.experimental.pallas import tpu as pltpu

_N_CAND = 8
_ROWS = 2 * 2048
_D = 1024
_NCHUNKS = 8
_CHUNK = _ROWS // _NCHUNKS


def _dma_body(s_ref, in_ref, out_ref, buf, sin, sout):
    s = s_ref[0]

    def ic(i):
        return pltpu.make_async_copy(
            in_ref.at[s, pl.ds(i * _CHUNK, _CHUNK), :],
            buf.at[pl.ds(i * _CHUNK, _CHUNK), :],
            sin.at[i],
        )

    def oc(i):
        return pltpu.make_async_copy(
            buf.at[pl.ds(i * _CHUNK, _CHUNK), :],
            out_ref.at[pl.ds(i * _CHUNK, _CHUNK), :],
            sout.at[i],
        )

    for i in range(_NCHUNKS):
        ic(i).start()
    for i in range(_NCHUNKS):
        ic(i).wait()
        oc(i).start()
    for i in range(_NCHUNKS):
        oc(i).wait()


def kernel(inputs, binary_gates, alpha, sampled):
    del binary_gates, alpha
    s = jnp.asarray(sampled, dtype=jnp.int32).reshape((1,))
    flat = inputs.reshape(_N_CAND, _ROWS, _D)
    out = pl.pallas_call(
        _dma_body,
        grid_spec=pltpu.PrefetchScalarGridSpec(
            num_scalar_prefetch=1,
            in_specs=[pl.BlockSpec(memory_space=pl.ANY)],
            out_specs=pl.BlockSpec(memory_space=pl.ANY),
            scratch_shapes=[
                pltpu.VMEM((_ROWS, _D), jnp.float32),
                pltpu.SemaphoreType.DMA((_NCHUNKS,)),
                pltpu.SemaphoreType.DMA((_NCHUNKS,)),
            ],
        ),
        out_shape=jax.ShapeDtypeStruct((_ROWS, _D), jnp.float32),
    )(s, flat)
    return out.reshape(2, 2048, _D)
